# baseline (speedup 1.0000x reference)
import functools

import jax
import jax.numpy as jnp
from jax import lax
from jax.experimental import pallas as pl
from jax.experimental.pallas import tpu as pltpu
from jax.experimental.pallas import tpu_sc as plsc

N_OBJS = 100000
EMBED_DIM = 32
B_ROWS = 16384
SEQ = 50
B_TOTAL = B_ROWS * SEQ
LANES = 128

_info = plsc.get_sparse_core_info()
NC, NS = _info.num_cores, _info.num_subcores
NW = NC * NS
B_PER_W = B_TOTAL // NW
ROWS_PER_W = B_ROWS // NW
CHUNK_ROWS = 32
CHUNK = CHUNK_ROWS * SEQ
CHUNKS = ROWS_PER_W // CHUNK_ROWS
NBUF = 2
DEPAD_PER_TILE = N_OBJS // NS

TPAD_BLK = 800
TPAD_GRID = N_OBJS // TPAD_BLK


def _tpad_body(t_ref, o_ref):
    x = t_ref[...]
    o_ref[...] = jnp.concatenate(
        [x, jnp.zeros((TPAD_BLK, LANES - EMBED_DIM), jnp.float32)], axis=1)


_tpad = pl.pallas_call(
    _tpad_body,
    grid=(TPAD_GRID,),
    in_specs=[pl.BlockSpec((TPAD_BLK, EMBED_DIM), lambda i: (i, 0))],
    out_specs=pl.BlockSpec((TPAD_BLK, LANES), lambda i: (i, 0)),
    out_shape=jax.ShapeDtypeStruct((N_OBJS, LANES), jnp.float32),
)

_mesh = plsc.VectorSubcoreMesh(core_axis_name="c", subcore_axis_name="s")


@functools.partial(
    pl.kernel,
    mesh=_mesh,
    out_type=(
        jax.ShapeDtypeStruct((B_ROWS, 56, 128), jnp.float32),
        jax.ShapeDtypeStruct((NC, N_OBJS, EMBED_DIM), jnp.float32),
    ),
    scratch_types=[
        [pltpu.VMEM((CHUNK,), jnp.int32) for _ in range(NBUF)],
        [pltpu.VMEM((CHUNK, EMBED_DIM), jnp.float32) for _ in range(NBUF)],
        [pltpu.SemaphoreType.DMA for _ in range(NBUF)],
        [pltpu.SemaphoreType.DMA for _ in range(NBUF)],
        [pltpu.SemaphoreType.DMA for _ in range(NBUF)],
        pltpu.SemaphoreType.DMA,
    ],
    compiler_params=pltpu.CompilerParams(use_tc_tiling_on_sc=False),
)
def _gather_kernel(tpad_hbm, idx_hbm, out_hbm, scr_hbm,
                   idx_v, rows_v, si, sg, so, sd):
    cidx = lax.axis_index("c")
    sid = lax.axis_index("s")
    wid = sid * NC + cidx
    wbase = wid * B_PER_W
    wrow = wid * ROWS_PER_W

    d0 = sid * DEPAD_PER_TILE
    pltpu.async_copy(
        tpad_hbm.at[pl.ds(d0, DEPAD_PER_TILE), pl.ds(0, EMBED_DIM)],
        scr_hbm.at[cidx, pl.ds(d0, DEPAD_PER_TILE)],
        sd)
    pltpu.make_async_copy(
        tpad_hbm.at[pl.ds(d0, DEPAD_PER_TILE), pl.ds(0, EMBED_DIM)],
        scr_hbm.at[cidx, pl.ds(d0, DEPAD_PER_TILE)],
        sd).wait()
    plsc.subcore_barrier()

    table_hbm = scr_hbm.at[cidx]

    def start_idx(c, b):
        base = wbase + c * CHUNK
        pltpu.async_copy(idx_hbm.at[pl.ds(base, CHUNK)], idx_v[b], si[b])

    def start_gather(b):
        pltpu.async_copy(table_hbm.at[idx_v[b]], rows_v[b], sg[b])

    def wait_gather(b):
        pltpu.make_async_copy(table_hbm.at[idx_v[b]], rows_v[b], sg[b]).wait()

    def start_out(c, b):
        row0 = wrow + c * CHUNK_ROWS
        for j in range(CHUNK_ROWS):
            pltpu.async_copy(
                rows_v[b].at[pl.ds(j * SEQ, SEQ)],
                out_hbm.at[row0 + j, pl.ds(0, SEQ), pl.ds(0, EMBED_DIM)],
                so[b])

    def wait_out(b):
        for _ in range(CHUNK_ROWS):
            pltpu.make_async_copy(
                rows_v[b].at[pl.ds(0, SEQ)],
                out_hbm.at[0, pl.ds(0, SEQ), pl.ds(0, EMBED_DIM)],
                so[b]).wait()

    start_idx(0, 0)
    start_idx(1, 1)
    pltpu.make_async_copy(idx_hbm.at[pl.ds(0, CHUNK)], idx_v[0], si[0]).wait()
    start_gather(0)
    for c in range(CHUNKS):
        b = c % NBUF
        nb = (c + 1) % NBUF
        if c + 1 < CHUNKS:
            pltpu.make_async_copy(
                idx_hbm.at[pl.ds(0, CHUNK)], idx_v[nb], si[nb]).wait()
            if c + 1 >= NBUF:
                wait_out(nb)
            start_gather(nb)
        wait_gather(b)
        start_out(c, b)
        if c + NBUF < CHUNKS:
            start_idx(c + NBUF, b)
    for b in range(NBUF):
        wait_out(b)


def kernel(x, table):
    idx = x.reshape(-1).astype(jnp.int32)
    out56, _ = _gather_kernel(_tpad(table), idx)
    return out56[:, :SEQ, :EMBED_DIM].reshape(x.shape + (EMBED_DIM,))
